# Initial kernel scaffold; baseline (speedup 1.0000x reference)
import jax
import jax.numpy as jnp
from jax.experimental import pallas as pl


def kernel(adj_data, u_i, u_j, graph_label, label, idx_mask, emb, H_v, phi, W1, b1, W2, b2, W3, b3, theta):
    raise NotImplementedError("write your pallas kernel here")


# speedup vs baseline: 1.3017x; 1.3017x over previous
import functools
import jax
import jax.numpy as jnp
from jax.experimental import pallas as pl
from jax.experimental.pallas import tpu as pltpu

_N = 10000
_VIEWS = 2
_D = 64
_BR = 200


def _attn_body(adj_ref, hv_ref, embT_ref, emb_ref, out_ref):
    adj = adj_ref[0]
    hv = hv_ref[0]
    vrow = jnp.dot(hv, embT_ref[...], preferred_element_type=jnp.float32)
    mask = adj > 0.0
    scores = jnp.where(mask, adj * vrow, -1e9)
    m = jnp.max(scores, axis=1, keepdims=True)
    e = jnp.where(mask, jnp.exp(scores - m), 0.0)
    denom = jnp.sum(e, axis=1, keepdims=True)
    inv = jnp.where(denom > 0.0, 1.0 / denom, 0.0)
    h = jnp.dot(e, emb_ref[...], preferred_element_type=jnp.float32)
    out_ref[0] = h * inv


def _attention(adj_data, emb, H_v):
    embT = emb.T
    hv3 = H_v.reshape(_VIEWS, 1, _D)
    grid = (_VIEWS, _N // _BR)
    return pl.pallas_call(
        _attn_body,
        grid=grid,
        in_specs=[
            pl.BlockSpec((1, _BR, _N), lambda v, i: (v, i, 0)),
            pl.BlockSpec((1, 1, _D), lambda v, i: (v, 0, 0)),
            pl.BlockSpec((_D, _N), lambda v, i: (0, 0)),
            pl.BlockSpec((_N, _D), lambda v, i: (0, 0)),
        ],
        out_specs=pl.BlockSpec((1, _BR, _D), lambda v, i: (v, i, 0)),
        out_shape=jax.ShapeDtypeStruct((_VIEWS, _N, _D), jnp.float32),
    )(adj_data, hv3, embT, emb)


def _dense_body(h1_ref, phi_ref, w1_ref, b1_ref, w2_ref, b2_ref, w3_ref,
                b3_ref, th_ref, lab_ref, au_ref, t1_ref, match_ref):
    h0 = h1_ref[0]
    h1v = h1_ref[1]
    p = phi_ref[...]
    pm = jnp.max(p, axis=1, keepdims=True)
    pe = jnp.exp(p - pm)
    w = pe / jnp.sum(pe, axis=1, keepdims=True)
    w0 = w[0:1, 0:1]
    w1 = w[0:1, 1:2]
    x = (jnp.dot(h0 * w0, w1_ref[0:_D, :], preferred_element_type=jnp.float32)
         + jnp.dot(h1v * w1, w1_ref[_D:2 * _D, :], preferred_element_type=jnp.float32)
         + b1_ref[...])
    x = jnp.dot(x, w2_ref[...], preferred_element_type=jnp.float32) + b2_ref[...]
    au = jnp.dot(x, w3_ref[...], preferred_element_type=jnp.float32) + b3_ref[...]
    au_ref[...] = au
    logits_raw = jnp.dot(au, th_ref[...], preferred_element_type=jnp.float32)
    s1 = jax.nn.softmax(logits_raw, axis=-1)
    s2 = jax.nn.softmax(s1, axis=-1)
    lab = lab_ref[...]
    t1_ref[...] = jnp.sum(lab * jnp.log(s2), axis=1, keepdims=True)
    am_l = s1[:, 1:2] > s1[:, 0:1]
    am_y = lab[:, 1:2] > lab[:, 0:1]
    match_ref[...] = (am_l == am_y).astype(jnp.float32)


def _dense(h1, phi, W1, b1, W2, b2, W3, b3, theta, label):
    phi_r = phi.reshape(1, _VIEWS)
    b1r = b1.reshape(1, -1)
    b2r = b2.reshape(1, -1)
    b3r = b3.reshape(1, -1)
    full = lambda s: pl.BlockSpec(s, lambda: tuple(0 for _ in s))
    return pl.pallas_call(
        _dense_body,
        in_specs=[full((_VIEWS, _N, _D)), full((1, _VIEWS)),
                  full(W1.shape), full(b1r.shape), full(W2.shape),
                  full(b2r.shape), full(W3.shape), full(b3r.shape),
                  full(theta.shape), full(label.shape)],
        out_specs=[full((_N, 32)), full((_N, 1)), full((_N, 1))],
        out_shape=[jax.ShapeDtypeStruct((_N, 32), jnp.float32),
                   jax.ShapeDtypeStruct((_N, 1), jnp.float32),
                   jax.ShapeDtypeStruct((_N, 1), jnp.float32)],
    )(h1, phi_r, W1, b1r, W2, b2r, W3, b3r, theta, label)


def kernel(adj_data, u_i, u_j, graph_label, label, idx_mask, emb, H_v, phi,
           W1, b1, W2, b2, W3, b3, theta):
    h1 = _attention(adj_data, emb, H_v)
    a_u, t1, match = _dense(h1, phi, W1, b1, W2, b2, W3, b3, theta, label)
    n_mask = idx_mask.shape[0]
    n_pairs = u_i.shape[0]
    loss1 = -(1.0 / n_mask) * jnp.sum(jnp.take(t1[:, 0], idx_mask))
    acc = jnp.mean(jnp.take(match[:, 0], idx_mask))
    ui_e = jnp.take(a_u, u_i, axis=0)
    uj_e = jnp.take(a_u, u_j, axis=0)
    ip = jnp.sum(ui_e * uj_e, axis=1)
    loss2 = -jnp.mean(jax.nn.log_sigmoid(graph_label * ip))
    loss = 0.5 * loss1 + 0.5 * loss2
    return loss, acc


# speedup vs baseline: 1.4726x; 1.1313x over previous
import functools
import jax
import jax.numpy as jnp
from jax.experimental import pallas as pl
from jax.experimental.pallas import tpu as pltpu

_N = 10000
_VIEWS = 2
_D = 64
_BR = 200


def _attn_body(adj_ref, hv_ref, embT_ref, embx_ref, out_ref):
    adj = adj_ref[0]
    hv2 = hv_ref[0] * 1.4426950408889634
    vrow2 = jnp.dot(hv2, embT_ref[...], preferred_element_type=jnp.float32)
    e = jnp.exp2(adj * vrow2)
    e = jnp.where(adj > 0.0, e, 0.0).astype(jnp.bfloat16)
    hf = jnp.dot(e, embx_ref[...], preferred_element_type=jnp.float32)
    denom = hf[:, _D:_D + 1]
    inv = jnp.where(denom > 0.0, 1.0 / denom, 0.0)
    out_ref[0] = hf[:, 0:_D] * inv


def _attention(adj_data, emb, H_v):
    embT = emb.T
    embx = jnp.concatenate(
        [emb, jnp.ones((_N, 1), jnp.float32)], axis=1).astype(jnp.bfloat16)
    hv3 = H_v.reshape(_VIEWS, 1, _D)
    grid = (_VIEWS, _N // _BR)
    return pl.pallas_call(
        _attn_body,
        grid=grid,
        in_specs=[
            pl.BlockSpec((1, _BR, _N), lambda v, i: (v, i, 0)),
            pl.BlockSpec((1, 1, _D), lambda v, i: (v, 0, 0)),
            pl.BlockSpec((_D, _N), lambda v, i: (0, 0)),
            pl.BlockSpec((_N, _D + 1), lambda v, i: (0, 0)),
        ],
        out_specs=pl.BlockSpec((1, _BR, _D), lambda v, i: (v, i, 0)),
        out_shape=jax.ShapeDtypeStruct((_VIEWS, _N, _D), jnp.float32),
    )(adj_data, hv3, embT, embx)


def _dense_body(h1_ref, phi_ref, w1_ref, b1_ref, w2_ref, b2_ref, w3_ref,
                b3_ref, th_ref, lab_ref, au_ref, t1_ref, match_ref):
    h0 = h1_ref[0]
    h1v = h1_ref[1]
    p = phi_ref[...]
    pm = jnp.max(p, axis=1, keepdims=True)
    pe = jnp.exp(p - pm)
    w = pe / jnp.sum(pe, axis=1, keepdims=True)
    w0 = w[0:1, 0:1]
    w1 = w[0:1, 1:2]
    x = (jnp.dot(h0 * w0, w1_ref[0:_D, :], preferred_element_type=jnp.float32)
         + jnp.dot(h1v * w1, w1_ref[_D:2 * _D, :], preferred_element_type=jnp.float32)
         + b1_ref[...])
    x = jnp.dot(x, w2_ref[...], preferred_element_type=jnp.float32) + b2_ref[...]
    au = jnp.dot(x, w3_ref[...], preferred_element_type=jnp.float32) + b3_ref[...]
    au_ref[...] = au
    logits_raw = jnp.dot(au, th_ref[...], preferred_element_type=jnp.float32)
    s1 = jax.nn.softmax(logits_raw, axis=-1)
    s2 = jax.nn.softmax(s1, axis=-1)
    lab = lab_ref[...]
    t1_ref[...] = jnp.sum(lab * jnp.log(s2), axis=1, keepdims=True)
    am_l = s1[:, 1:2] > s1[:, 0:1]
    am_y = lab[:, 1:2] > lab[:, 0:1]
    match_ref[...] = (am_l == am_y).astype(jnp.float32)


def _dense(h1, phi, W1, b1, W2, b2, W3, b3, theta, label):
    phi_r = phi.reshape(1, _VIEWS)
    b1r = b1.reshape(1, -1)
    b2r = b2.reshape(1, -1)
    b3r = b3.reshape(1, -1)
    full = lambda s: pl.BlockSpec(s, lambda: tuple(0 for _ in s))
    return pl.pallas_call(
        _dense_body,
        in_specs=[full((_VIEWS, _N, _D)), full((1, _VIEWS)),
                  full(W1.shape), full(b1r.shape), full(W2.shape),
                  full(b2r.shape), full(W3.shape), full(b3r.shape),
                  full(theta.shape), full(label.shape)],
        out_specs=[full((_N, 32)), full((_N, 1)), full((_N, 1))],
        out_shape=[jax.ShapeDtypeStruct((_N, 32), jnp.float32),
                   jax.ShapeDtypeStruct((_N, 1), jnp.float32),
                   jax.ShapeDtypeStruct((_N, 1), jnp.float32)],
    )(h1, phi_r, W1, b1r, W2, b2r, W3, b3r, theta, label)


def kernel(adj_data, u_i, u_j, graph_label, label, idx_mask, emb, H_v, phi,
           W1, b1, W2, b2, W3, b3, theta):
    h1 = _attention(adj_data, emb, H_v)
    a_u, t1, match = _dense(h1, phi, W1, b1, W2, b2, W3, b3, theta, label)
    n_mask = idx_mask.shape[0]
    n_pairs = u_i.shape[0]
    loss1 = -(1.0 / n_mask) * jnp.sum(jnp.take(t1[:, 0], idx_mask))
    acc = jnp.mean(jnp.take(match[:, 0], idx_mask))
    ui_e = jnp.take(a_u, u_i, axis=0)
    uj_e = jnp.take(a_u, u_j, axis=0)
    ip = jnp.sum(ui_e * uj_e, axis=1)
    loss2 = -jnp.mean(jax.nn.log_sigmoid(graph_label * ip))
    loss = 0.5 * loss1 + 0.5 * loss2
    return loss, acc


# speedup vs baseline: 1.4954x; 1.0154x over previous
import functools
import jax
import jax.numpy as jnp
from jax.experimental import pallas as pl
from jax.experimental.pallas import tpu as pltpu

_N = 10000
_VIEWS = 2
_D = 64
_BR = 400


def _attn_body(adj_ref, hv_ref, embT_ref, embx_ref, out_ref):
    adj = adj_ref[0]
    hv2 = hv_ref[0] * 1.4426950408889634
    vrow2 = jnp.dot(hv2, embT_ref[...], preferred_element_type=jnp.float32)
    e = jnp.exp2(adj * vrow2)
    e = jnp.where(adj > 0.0, e, 0.0).astype(jnp.bfloat16)
    hf = jnp.dot(e, embx_ref[...], preferred_element_type=jnp.float32)
    denom = hf[:, _D:_D + 1]
    inv = jnp.where(denom > 0.0, 1.0 / denom, 0.0)
    out_ref[0] = hf[:, 0:_D] * inv


def _attention(adj_data, emb, H_v):
    embT = emb.T
    embx = jnp.concatenate(
        [emb, jnp.ones((_N, 1), jnp.float32)], axis=1).astype(jnp.bfloat16)
    hv3 = H_v.reshape(_VIEWS, 1, _D)
    grid = (_VIEWS, _N // _BR)
    return pl.pallas_call(
        _attn_body,
        grid=grid,
        in_specs=[
            pl.BlockSpec((1, _BR, _N), lambda v, i: (v, i, 0)),
            pl.BlockSpec((1, 1, _D), lambda v, i: (v, 0, 0)),
            pl.BlockSpec((_D, _N), lambda v, i: (0, 0)),
            pl.BlockSpec((_N, _D + 1), lambda v, i: (0, 0)),
        ],
        out_specs=pl.BlockSpec((1, _BR, _D), lambda v, i: (v, i, 0)),
        out_shape=jax.ShapeDtypeStruct((_VIEWS, _N, _D), jnp.float32),
    )(adj_data, hv3, embT, embx)


def _dense_body(h1_ref, phi_ref, w1_ref, b1_ref, w2_ref, b2_ref, w3_ref,
                b3_ref, th_ref, lab_ref, au_ref, t1_ref, match_ref):
    h0 = h1_ref[0]
    h1v = h1_ref[1]
    p = phi_ref[...]
    pm = jnp.max(p, axis=1, keepdims=True)
    pe = jnp.exp(p - pm)
    w = pe / jnp.sum(pe, axis=1, keepdims=True)
    w0 = w[0:1, 0:1]
    w1 = w[0:1, 1:2]
    x = (jnp.dot(h0 * w0, w1_ref[0:_D, :], preferred_element_type=jnp.float32)
         + jnp.dot(h1v * w1, w1_ref[_D:2 * _D, :], preferred_element_type=jnp.float32)
         + b1_ref[...])
    x = jnp.dot(x, w2_ref[...], preferred_element_type=jnp.float32) + b2_ref[...]
    au = jnp.dot(x, w3_ref[...], preferred_element_type=jnp.float32) + b3_ref[...]
    au_ref[...] = au
    logits_raw = jnp.dot(au, th_ref[...], preferred_element_type=jnp.float32)
    s1 = jax.nn.softmax(logits_raw, axis=-1)
    s2 = jax.nn.softmax(s1, axis=-1)
    lab = lab_ref[...]
    t1_ref[...] = jnp.sum(lab * jnp.log(s2), axis=1, keepdims=True)
    am_l = s1[:, 1:2] > s1[:, 0:1]
    am_y = lab[:, 1:2] > lab[:, 0:1]
    match_ref[...] = (am_l == am_y).astype(jnp.float32)


def _dense(h1, phi, W1, b1, W2, b2, W3, b3, theta, label):
    phi_r = phi.reshape(1, _VIEWS)
    b1r = b1.reshape(1, -1)
    b2r = b2.reshape(1, -1)
    b3r = b3.reshape(1, -1)
    full = lambda s: pl.BlockSpec(s, lambda: tuple(0 for _ in s))
    return pl.pallas_call(
        _dense_body,
        in_specs=[full((_VIEWS, _N, _D)), full((1, _VIEWS)),
                  full(W1.shape), full(b1r.shape), full(W2.shape),
                  full(b2r.shape), full(W3.shape), full(b3r.shape),
                  full(theta.shape), full(label.shape)],
        out_specs=[full((_N, 32)), full((_N, 1)), full((_N, 1))],
        out_shape=[jax.ShapeDtypeStruct((_N, 32), jnp.float32),
                   jax.ShapeDtypeStruct((_N, 1), jnp.float32),
                   jax.ShapeDtypeStruct((_N, 1), jnp.float32)],
    )(h1, phi_r, W1, b1r, W2, b2r, W3, b3r, theta, label)


def kernel(adj_data, u_i, u_j, graph_label, label, idx_mask, emb, H_v, phi,
           W1, b1, W2, b2, W3, b3, theta):
    h1 = _attention(adj_data, emb, H_v)
    a_u, t1, match = _dense(h1, phi, W1, b1, W2, b2, W3, b3, theta, label)
    n_mask = idx_mask.shape[0]
    n_pairs = u_i.shape[0]
    loss1 = -(1.0 / n_mask) * jnp.sum(jnp.take(t1[:, 0], idx_mask))
    acc = jnp.mean(jnp.take(match[:, 0], idx_mask))
    ui_e = jnp.take(a_u, u_i, axis=0)
    uj_e = jnp.take(a_u, u_j, axis=0)
    ip = jnp.sum(ui_e * uj_e, axis=1)
    loss2 = -jnp.mean(jax.nn.log_sigmoid(graph_label * ip))
    loss = 0.5 * loss1 + 0.5 * loss2
    return loss, acc


# speedup vs baseline: 4.8122x; 3.2181x over previous
import functools
import jax
import jax.numpy as jnp
from jax import lax
from jax.experimental import pallas as pl
from jax.experimental.pallas import tpu as pltpu
from jax.experimental.pallas import tpu_sc as plsc

_N = 10000
_VIEWS = 2
_D = 64
_BSUBU = 10
_NSUB = 5
_BSUB = _BSUBU * 8
_BR = _BSUB * _NSUB


def _attn_body(*refs):
    adj_refs = refs[:_NSUB]
    hv_ref, embT_ref, embx_ref, out_ref = refs[_NSUB:]
    hv2 = hv_ref[0] * 1.4426950408889634
    vrow2 = jnp.dot(hv2, embT_ref[...], preferred_element_type=jnp.float32)
    embx = embx_ref[...]
    for k in range(_NSUB):
        adj = adj_refs[k][0].reshape(_BSUB, _N)
        e = jnp.exp2(adj * vrow2)
        e = jnp.where(adj > 0.0, e, 0.0).astype(jnp.bfloat16)
        hf = jnp.dot(e, embx, preferred_element_type=jnp.float32)
        denom = hf[:, _D:_D + 1]
        inv = jnp.where(denom > 0.0, 1.0 / denom, 0.0)
        out_ref[0, k * _BSUB:(k + 1) * _BSUB, :] = hf[:, 0:_D] * inv


def _attention(adj_data, emb, H_v):
    adj4 = adj_data.reshape(_VIEWS, _N // 8, 8, _N)
    embT = emb.T
    embx = jnp.concatenate(
        [emb, jnp.ones((_N, 1), jnp.float32)], axis=1).astype(jnp.bfloat16)
    hv3 = H_v.reshape(_VIEWS, 1, _D)
    grid = (_VIEWS, _N // _BR)
    adj_specs = [
        pl.BlockSpec((1, _BSUBU, 8, _N),
                     functools.partial(
                         lambda k, v, i: (v, i * _NSUB + k, 0, 0), k))
        for k in range(_NSUB)
    ]
    return pl.pallas_call(
        _attn_body,
        grid=grid,
        in_specs=adj_specs + [
            pl.BlockSpec((1, 1, _D), lambda v, i: (v, 0, 0)),
            pl.BlockSpec((_D, _N), lambda v, i: (0, 0)),
            pl.BlockSpec((_N, _D + 1), lambda v, i: (0, 0)),
        ],
        out_specs=pl.BlockSpec((1, _BR, _D), lambda v, i: (v, i, 0)),
        out_shape=jax.ShapeDtypeStruct((_VIEWS, _N, _D), jnp.float32),
    )(*([adj4] * _NSUB), hv3, embT, embx)


_E3 = 32
_NPAD = _N + 16


def _dense_body(h1_ref, phi_ref, w1_ref, b1_ref, w2_ref, b2_ref, w3_ref,
                b3_ref, th_ref, lab_ref, auT_ref, t1_ref, match_ref):
    h0 = h1_ref[0]
    h1v = h1_ref[1]
    p = phi_ref[...]
    pm = jnp.max(p, axis=1, keepdims=True)
    pe = jnp.exp(p - pm)
    w = pe / jnp.sum(pe, axis=1, keepdims=True)
    w0 = w[0:1, 0:1]
    w1 = w[0:1, 1:2]
    x = (jnp.dot(h0 * w0, w1_ref[0:_D, :], preferred_element_type=jnp.float32)
         + jnp.dot(h1v * w1, w1_ref[_D:2 * _D, :], preferred_element_type=jnp.float32)
         + b1_ref[...])
    x = jnp.dot(x, w2_ref[...], preferred_element_type=jnp.float32) + b2_ref[...]
    au = jnp.dot(x, w3_ref[...], preferred_element_type=jnp.float32) + b3_ref[...]
    auT_ref[...] = jnp.concatenate(
        [jnp.transpose(au), jnp.zeros((_E3, 16), jnp.float32)], axis=1)
    logits_raw = jnp.dot(au, th_ref[...], preferred_element_type=jnp.float32)
    s1 = jax.nn.softmax(logits_raw, axis=-1)
    s2 = jax.nn.softmax(s1, axis=-1)
    lab = lab_ref[...]
    t1v = jnp.sum(lab * jnp.log(s2), axis=1)
    t1_ref[...] = jnp.concatenate([t1v, jnp.zeros((16,), jnp.float32)])
    am = ((s1[:, 1:2] > s1[:, 0:1]) ==
          (lab[:, 1:2] > lab[:, 0:1])).astype(jnp.float32)[:, 0]
    match_ref[...] = jnp.concatenate([am, jnp.zeros((16,), jnp.float32)])


def _dense(h1, phi, W1, b1, W2, b2, W3, b3, theta, label):
    phi_r = phi.reshape(1, _VIEWS)
    b1r = b1.reshape(1, -1)
    b2r = b2.reshape(1, -1)
    b3r = b3.reshape(1, -1)
    full = lambda s: pl.BlockSpec(s, lambda: tuple(0 for _ in s))
    return pl.pallas_call(
        _dense_body,
        in_specs=[full((_VIEWS, _N, _D)), full((1, _VIEWS)),
                  full(W1.shape), full(b1r.shape), full(W2.shape),
                  full(b2r.shape), full(W3.shape), full(b3r.shape),
                  full(theta.shape), full(label.shape)],
        out_specs=[full((_E3, _NPAD)), full((_NPAD,)), full((_NPAD,))],
        out_shape=[jax.ShapeDtypeStruct((_E3, _NPAD), jnp.float32),
                   jax.ShapeDtypeStruct((_NPAD,), jnp.float32),
                   jax.ShapeDtypeStruct((_NPAD,), jnp.float32)],
    )(h1, phi_r, W1, b1r, W2, b2r, W3, b3r, theta, label)


_NFG = 4
_NPG = 8
_NPAIRS = 160000
_NPPAD = 163840
_PPG = _NPPAD // _NPG
_PCH = 4096
_NCH = _PPG // _PCH
_MASKPAD = 8192
_MPW = _MASKPAD // 32


def _sc_body(auT_hbm, ui_hbm, uj_hbm, idxp_hbm, t1_hbm, mt_hbm,
             ip_hbm, t1a_hbm, ma_hbm,
             part_v, ui_v, uj_v, ip_v, t1_v, mt_v, idx_v, a1_v, a2_v):
    wid = lax.axis_index("s") * 2 + lax.axis_index("c")
    fg = lax.rem(wid, _NFG)
    pg = lax.div(wid, _NFG)
    row0 = pl.multiple_of(fg * (8 * _NPAD), 8 * _NPAD)
    pltpu.sync_copy(auT_hbm.at[pl.ds(row0, 8 * _NPAD)], part_v)
    for c in range(_NCH):
        off = pl.multiple_of(pg * _PPG + c * _PCH, _PCH)
        pltpu.sync_copy(ui_hbm.at[pl.ds(off, _PCH)], ui_v)
        pltpu.sync_copy(uj_hbm.at[pl.ds(off, _PCH)], uj_v)

        def chunk_body(s, carry):
            ui16 = ui_v[pl.ds(s * 16, 16)]
            uj16 = uj_v[pl.ds(s * 16, 16)]
            acc = jnp.zeros((16,), jnp.float32)
            for f in range(8):
                fo = jnp.int32(f * _NPAD)
                gi = plsc.load_gather(part_v, [ui16 + fo])
                gj = plsc.load_gather(part_v, [uj16 + fo])
                acc = acc + gi * gj
            ip_v[pl.ds(s * 16, 16)] = acc
            return carry

        lax.fori_loop(0, _PCH // 16, chunk_body, 0)
        pltpu.sync_copy(ip_v, ip_hbm.at[fg, 0, pl.ds(off, _PCH)])
    pltpu.sync_copy(t1_hbm, t1_v)
    pltpu.sync_copy(mt_hbm, mt_v)
    moff = pl.multiple_of(wid * _MPW, _MPW)
    pltpu.sync_copy(idxp_hbm.at[pl.ds(moff, _MPW)], idx_v)
    acc1 = jnp.zeros((16,), jnp.float32)
    acc2 = jnp.zeros((16,), jnp.float32)
    for s in range(_MPW // 16):
        i16 = idx_v[pl.ds(s * 16, 16)]
        acc1 = acc1 + plsc.load_gather(t1_v, [i16])
        acc2 = acc2 + plsc.load_gather(mt_v, [i16])
    a1_v[...] = acc1
    a2_v[...] = acc2
    pltpu.sync_copy(a1_v, t1a_hbm.at[wid, 0])
    pltpu.sync_copy(a2_v, ma_hbm.at[wid, 0])


def _sc_gather(auT, uip, ujp, idxp, t1ext, mtext):
    kfn = pl.kernel(
        _sc_body,
        out_type=[jax.ShapeDtypeStruct((_NFG, 1, _NPPAD), jnp.float32),
                  jax.ShapeDtypeStruct((32, 1, 16), jnp.float32),
                  jax.ShapeDtypeStruct((32, 1, 16), jnp.float32)],
        mesh=plsc.VectorSubcoreMesh(core_axis_name="c", subcore_axis_name="s"),
        compiler_params=pltpu.CompilerParams(needs_layout_passes=False),
        scratch_types=[
            pltpu.VMEM((8 * _NPAD,), jnp.float32),
            pltpu.VMEM((_PCH,), jnp.int32),
            pltpu.VMEM((_PCH,), jnp.int32),
            pltpu.VMEM((_PCH,), jnp.float32),
            pltpu.VMEM((_NPAD,), jnp.float32),
            pltpu.VMEM((_NPAD,), jnp.float32),
            pltpu.VMEM((_MPW,), jnp.int32),
            pltpu.VMEM((16,), jnp.float32),
            pltpu.VMEM((16,), jnp.float32),
        ])
    return kfn(auT, uip, ujp, idxp, t1ext, mtext)


def _loss_body(ipp_ref, gl_ref, wm_ref, t1a_ref, ma_ref, loss_ref, acc_ref):
    ip = jnp.sum(ipp_ref[...], axis=0)
    z = gl_ref[...] * ip
    ls = jnp.minimum(z, 0.0) - jnp.log(1.0 + jnp.exp(-jnp.abs(z)))
    loss2 = -jnp.sum(ls * wm_ref[...]) / float(_NPAIRS)
    loss1 = -jnp.sum(t1a_ref[...]) / 5000.0
    accv = jnp.sum(ma_ref[...]) / 5000.0
    loss_ref[...] = (0.5 * loss1 + 0.5 * loss2).reshape(1, 1)
    acc_ref[...] = accv.reshape(1, 1)


def _losses(ipp, glp, wmask, t1a, ma):
    ipp3 = ipp.reshape(_NFG, _NPPAD // 128, 128)
    gl2 = glp.reshape(_NPPAD // 128, 128)
    wm2 = wmask.reshape(_NPPAD // 128, 128)
    t1a2 = t1a.reshape(32, 16)
    ma2 = ma.reshape(32, 16)
    full = lambda s: pl.BlockSpec(s, lambda: tuple(0 for _ in s))
    return pl.pallas_call(
        _loss_body,
        in_specs=[full(ipp3.shape), full(gl2.shape), full(wm2.shape),
                  full((32, 16)), full((32, 16))],
        out_specs=[full((1, 1)), full((1, 1))],
        out_shape=[jax.ShapeDtypeStruct((1, 1), jnp.float32),
                   jax.ShapeDtypeStruct((1, 1), jnp.float32)],
    )(ipp3, gl2, wm2, t1a2, ma2)


def kernel(adj_data, u_i, u_j, graph_label, label, idx_mask, emb, H_v, phi,
           W1, b1, W2, b2, W3, b3, theta):
    h1 = _attention(adj_data, emb, H_v)
    auT, t1ext, mtext = _dense(h1, phi, W1, b1, W2, b2, W3, b3, theta, label)
    npad = _NPPAD - _NPAIRS
    uip = jnp.concatenate([u_i.astype(jnp.int32),
                           jnp.full((npad,), _N, jnp.int32)])
    ujp = jnp.concatenate([u_j.astype(jnp.int32),
                           jnp.full((npad,), _N, jnp.int32)])
    idxp = jnp.concatenate(
        [idx_mask.astype(jnp.int32),
         jnp.full((_MASKPAD - idx_mask.shape[0],), _N, jnp.int32)])
    glp = jnp.concatenate([graph_label, jnp.zeros((npad,), jnp.float32)])
    wmask = jnp.concatenate([jnp.ones((_NPAIRS,), jnp.float32),
                             jnp.zeros((npad,), jnp.float32)])
    ipp, t1a, ma = _sc_gather(auT.reshape(-1), uip, ujp, idxp, t1ext, mtext)
    loss, acc = _losses(ipp, glp, wmask, t1a, ma)
    return loss[0, 0], acc[0, 0]
